# baseline (speedup 1.0000x reference)
import functools

import jax
import jax.numpy as jnp
from jax.experimental import pallas as pl
from jax.experimental.pallas import tpu as pltpu
from jax.experimental.pallas import tpu_sc as plsc

CURV_SCALE = 0.1
GATHER_WINDOW = 256


def _gather_sc(table, idx_flat):
    n = idx_flat.shape[0]
    d = table.shape[1]
    mesh = plsc.VectorSubcoreMesh(core_axis_name="c", subcore_axis_name="s")
    idx2d = idx_flat.reshape(1, n)

    @functools.partial(
        pl.kernel,
        out_type=jax.ShapeDtypeStruct((n, d), table.dtype),
        mesh=mesh,
        compiler_params=pltpu.CompilerParams(use_tc_tiling_on_sc=False),
    )
    def gather_kernel(table_hbm, idx_hbm, out_hbm):
        def body(i_vmem, o_vmem):
            pltpu.sync_copy(table_hbm.at[i_vmem.at[0]], o_vmem)

        pltpu.emit_pipeline(
            body,
            grid=(n // GATHER_WINDOW,),
            in_specs=[pl.BlockSpec((1, GATHER_WINDOW), lambda i: (0, i))],
            out_specs=[pl.BlockSpec((GATHER_WINDOW, d), lambda i: (i, 0))],
            core_axis_name=("c", "s"),
            dimension_semantics=(pltpu.PARALLEL,),
        )(idx_hbm, out_hbm)

    return gather_kernel(table, idx2d)


def _pack_table_tc(tab_t, vocab, d):
    pack = 128 // d
    blk = 4096
    bq = blk // pack
    ng = -(-vocab // blk)
    rows_out = ng * bq

    def body(x_ref, o_ref):
        x = x_ref[...]
        z = jnp.concatenate(
            [x[:, a * bq:(a + 1) * bq] for a in range(pack)], axis=0
        )
        o_ref[...] = z.T

    return pl.pallas_call(
        body,
        grid=(ng,),
        in_specs=[pl.BlockSpec((d, blk), lambda i: (0, i))],
        out_specs=pl.BlockSpec((bq, 128), lambda i: (i, 0)),
        out_shape=jax.ShapeDtypeStruct((rows_out, 128), jnp.float32),
        compiler_params=pltpu.CompilerParams(
            dimension_semantics=("parallel",)
        ),
    )(tab_t)


def _project_tc(emb4, w_bd, b_col, seq, bsz, d):
    pack = 128 // d
    bq = bsz // pack

    def body(x_ref, w_ref, b_ref, e_ref, c_ref):
        x4t = x_ref[0].T
        raw = jnp.dot(w_ref[...], x4t, preferred_element_type=jnp.float32)
        y4 = jnp.tanh(raw + b_ref[...]) * CURV_SCALE
        for c in range(pack):
            e_ref[0, :, c * bq:(c + 1) * bq] = x4t[c * d:(c + 1) * d, :]
            c_ref[0, :, c * bq:(c + 1) * bq] = y4[c * d:(c + 1) * d, :]

    return pl.pallas_call(
        body,
        grid=(seq,),
        in_specs=[
            pl.BlockSpec((1, bq, 128), lambda i: (i, 0, 0)),
            pl.BlockSpec((pack * d, pack * d), lambda i: (0, 0)),
            pl.BlockSpec((pack * d, 1), lambda i: (0, 0)),
        ],
        out_specs=[
            pl.BlockSpec((1, d, bsz), lambda i: (i, 0, 0)),
            pl.BlockSpec((1, d, bsz), lambda i: (i, 0, 0)),
        ],
        out_shape=[
            jax.ShapeDtypeStruct((seq, d, bsz), jnp.float32),
            jax.ShapeDtypeStruct((seq, d, bsz), jnp.float32),
        ],
        compiler_params=pltpu.CompilerParams(
            dimension_semantics=("parallel",)
        ),
    )(emb4, w_bd, b_col)


def kernel(token_ids, table, W_curv, b_curv):
    bsz, seq = token_ids.shape
    vocab, d = table.shape
    pack = 128 // d
    idx = (
        token_ids.T.astype(jnp.int32)
        .reshape(seq, pack, bsz // pack)
        .transpose(0, 2, 1)
        .reshape(-1)
    )
    blk = 4096
    bq = blk // pack
    table4 = _pack_table_tc(table.T, vocab, d)
    u = idx % blk
    idx = pack * (bq * (idx // blk) + u % bq) + u // bq
    emb_lin = _gather_sc(table4.reshape(table4.shape[0] * pack, d), idx)
    emb4 = emb_lin.reshape(seq, (bsz * d) // 128, 128)
    w_bd = jnp.kron(jnp.eye(pack, dtype=W_curv.dtype), W_curv.T)
    b_col = jnp.tile(b_curv, pack).reshape(pack * d, 1)
    emb_t, curv_t = _project_tc(emb4, w_bd, b_col, seq, bsz, d)
    return (
        jnp.transpose(emb_t, (2, 0, 1)),
        jnp.transpose(curv_t, (2, 0, 1)),
    )


# speedup vs baseline: 1.2771x; 1.2771x over previous
import functools

import jax
import jax.numpy as jnp
from jax.experimental import pallas as pl
from jax.experimental.pallas import tpu as pltpu
from jax.experimental.pallas import tpu_sc as plsc

CURV_SCALE = 0.1
GATHER_WINDOW = 256


def _gather_sc(table, idx_flat):
    n = idx_flat.shape[0]
    d = table.shape[1]
    mesh = plsc.VectorSubcoreMesh(core_axis_name="c", subcore_axis_name="s")
    idx2d = idx_flat.reshape(n // 128, 128)
    rows_per_win = GATHER_WINDOW // 128

    @functools.partial(
        pl.kernel,
        out_type=jax.ShapeDtypeStruct((n, d), table.dtype),
        mesh=mesh,
        compiler_params=pltpu.CompilerParams(use_tc_tiling_on_sc=False),
    )
    def gather_kernel(table_hbm, idx_hbm, out_hbm):
        def body(i_vmem, o_vmem):
            for r in range(rows_per_win):
                pltpu.sync_copy(
                    table_hbm.at[i_vmem.at[r]],
                    o_vmem.at[pl.ds(r * 128, 128)],
                )

        pltpu.emit_pipeline(
            body,
            grid=(n // GATHER_WINDOW,),
            in_specs=[pl.BlockSpec((rows_per_win, 128), lambda i: (i, 0))],
            out_specs=[pl.BlockSpec((GATHER_WINDOW, d), lambda i: (i, 0))],
            core_axis_name=("c", "s"),
            dimension_semantics=(pltpu.PARALLEL,),
        )(idx_hbm, out_hbm)

    return gather_kernel(table, idx2d)


def _pack_table_tc(tab_t, vocab, d):
    pack = 128 // d
    blk = 8192
    bq = blk // pack
    ng = -(-vocab // blk)
    rows_out = ng * bq

    def body(x_ref, o_ref):
        x = x_ref[...]
        z = jnp.concatenate(
            [x[:, a * bq:(a + 1) * bq] for a in range(pack)], axis=0
        )
        o_ref[...] = z.T

    return pl.pallas_call(
        body,
        grid=(ng,),
        in_specs=[pl.BlockSpec((d, blk), lambda i: (0, i))],
        out_specs=pl.BlockSpec((bq, 128), lambda i: (i, 0)),
        out_shape=jax.ShapeDtypeStruct((rows_out, 128), jnp.float32),
        compiler_params=pltpu.CompilerParams(
            dimension_semantics=("parallel",)
        ),
    )(tab_t)


def _project_tc(emb4, w_bd, b_col, seq, bsz, d):
    pack = 128 // d
    bq = bsz // pack
    sstep = 2

    def body(x_ref, w_ref, b_ref, e_ref, c_ref):
        for t in range(sstep):
            x4t = x_ref[t].T
            raw = jnp.dot(
                w_ref[...], x4t, preferred_element_type=jnp.float32
            )
            y4 = jnp.tanh(raw + b_ref[...]) * CURV_SCALE
            for c in range(pack):
                e_ref[t, :, c * bq:(c + 1) * bq] = x4t[c * d:(c + 1) * d, :]
                c_ref[t, :, c * bq:(c + 1) * bq] = y4[c * d:(c + 1) * d, :]

    return pl.pallas_call(
        body,
        grid=(seq // sstep,),
        in_specs=[
            pl.BlockSpec((sstep, bq, 128), lambda i: (i, 0, 0)),
            pl.BlockSpec((pack * d, pack * d), lambda i: (0, 0)),
            pl.BlockSpec((pack * d, 1), lambda i: (0, 0)),
        ],
        out_specs=[
            pl.BlockSpec((sstep, d, bsz), lambda i: (i, 0, 0)),
            pl.BlockSpec((sstep, d, bsz), lambda i: (i, 0, 0)),
        ],
        out_shape=[
            jax.ShapeDtypeStruct((seq, d, bsz), jnp.float32),
            jax.ShapeDtypeStruct((seq, d, bsz), jnp.float32),
        ],
        compiler_params=pltpu.CompilerParams(
            dimension_semantics=("parallel",)
        ),
    )(emb4, w_bd, b_col)


def kernel(token_ids, table, W_curv, b_curv):
    bsz, seq = token_ids.shape
    vocab, d = table.shape
    pack = 128 // d
    idx = (
        token_ids.T.astype(jnp.int32)
        .reshape(seq, pack, bsz // pack)
        .transpose(0, 2, 1)
        .reshape(-1)
    )
    blk = 8192
    bq = blk // pack
    table4 = _pack_table_tc(table.T, vocab, d)
    u = idx % blk
    idx = pack * (bq * (idx // blk) + u % bq) + u // bq
    emb_lin = _gather_sc(table4.reshape(table4.shape[0] * pack, d), idx)
    emb4 = emb_lin.reshape(seq, (bsz * d) // 128, 128)
    w_bd = jnp.kron(jnp.eye(pack, dtype=W_curv.dtype), W_curv.T)
    b_col = jnp.tile(b_curv, pack).reshape(pack * d, 1)
    emb_t, curv_t = _project_tc(emb4, w_bd, b_col, seq, bsz, d)
    return (
        jnp.transpose(emb_t, (2, 0, 1)),
        jnp.transpose(curv_t, (2, 0, 1)),
    )


# speedup vs baseline: 1.5550x; 1.2176x over previous
import functools

import jax
import jax.numpy as jnp
from jax.experimental import pallas as pl
from jax.experimental.pallas import tpu as pltpu
from jax.experimental.pallas import tpu_sc as plsc

CURV_SCALE = 0.1
GATHER_WINDOW = 256


def _gather_sc(table, idx_flat, seq, bsz):
    n = idx_flat.shape[0]
    d = table.shape[1]
    pack = 128 // d
    bq = bsz // pack
    rows_per_win = bq // 128
    mesh = plsc.VectorSubcoreMesh(core_axis_name="c", subcore_axis_name="s")
    idx2d = idx_flat.reshape(n // 128, 128)

    @functools.partial(
        pl.kernel,
        out_type=jax.ShapeDtypeStruct((seq * bq, 128), table.dtype),
        mesh=mesh,
        compiler_params=pltpu.CompilerParams(use_tc_tiling_on_sc=False),
    )
    def gather_kernel(table_hbm, idx_hbm, out_hbm):
        def body(i_vmem, o_vmem):
            for r in range(rows_per_win):
                pltpu.sync_copy(
                    table_hbm.at[i_vmem.at[r]],
                    o_vmem.at[pl.ds(r * 128, 128)],
                )

        pltpu.emit_pipeline(
            body,
            grid=(seq, pack),
            in_specs=[
                pl.BlockSpec(
                    (rows_per_win, 128), lambda s, c: (s * pack + c, 0)
                )
            ],
            out_specs=[pl.BlockSpec((bq, d), lambda s, c: (s, c))],
            core_axis_name=("c", "s"),
            dimension_semantics=(pltpu.PARALLEL, pltpu.PARALLEL),
        )(idx_hbm, out_hbm)

    return gather_kernel(table, idx2d)


def _pack_table_tc(tab_t, vocab, d):
    pack = 128 // d
    blk = 8192
    bq = blk // pack
    ng = -(-vocab // blk)
    rows_out = ng * bq

    def body(x_ref, o_ref):
        x = x_ref[...]
        z = jnp.concatenate(
            [x[:, a * bq:(a + 1) * bq] for a in range(pack)], axis=0
        )
        o_ref[...] = z.T

    return pl.pallas_call(
        body,
        grid=(ng,),
        in_specs=[pl.BlockSpec((d, blk), lambda i: (0, i))],
        out_specs=pl.BlockSpec((bq, 128), lambda i: (i, 0)),
        out_shape=jax.ShapeDtypeStruct((rows_out, 128), jnp.float32),
        compiler_params=pltpu.CompilerParams(
            dimension_semantics=("parallel",)
        ),
    )(tab_t)


def _project_tc(emb4, w_bd, b_col, seq, bsz, d):
    pack = 128 // d
    bq = bsz // pack
    sstep = 2

    def body(x_ref, w_ref, b_ref, e_ref, c_ref):
        for t in range(sstep):
            x4t = x_ref[t].T
            raw = jnp.dot(
                w_ref[...], x4t, preferred_element_type=jnp.float32
            )
            y4 = jnp.tanh(raw + b_ref[...]) * CURV_SCALE
            for c in range(pack):
                e_ref[t, :, c * bq:(c + 1) * bq] = x4t[c * d:(c + 1) * d, :]
                c_ref[t, :, c * bq:(c + 1) * bq] = y4[c * d:(c + 1) * d, :]

    return pl.pallas_call(
        body,
        grid=(seq // sstep,),
        in_specs=[
            pl.BlockSpec((sstep, bq, 128), lambda i: (i, 0, 0)),
            pl.BlockSpec((pack * d, pack * d), lambda i: (0, 0)),
            pl.BlockSpec((pack * d, 1), lambda i: (0, 0)),
        ],
        out_specs=[
            pl.BlockSpec((sstep, d, bsz), lambda i: (i, 0, 0)),
            pl.BlockSpec((sstep, d, bsz), lambda i: (i, 0, 0)),
        ],
        out_shape=[
            jax.ShapeDtypeStruct((seq, d, bsz), jnp.float32),
            jax.ShapeDtypeStruct((seq, d, bsz), jnp.float32),
        ],
        compiler_params=pltpu.CompilerParams(
            dimension_semantics=("parallel",)
        ),
    )(emb4, w_bd, b_col)


def kernel(token_ids, table, W_curv, b_curv):
    bsz, seq = token_ids.shape
    vocab, d = table.shape
    pack = 128 // d
    idx = token_ids.T.astype(jnp.int32).reshape(-1)
    blk = 8192
    bq = blk // pack
    table4 = _pack_table_tc(table.T, vocab, d)
    u = idx % blk
    idx = pack * (bq * (idx // blk) + u % bq) + u // bq
    emb4 = _gather_sc(
        table4.reshape(table4.shape[0] * pack, d), idx, seq, bsz
    ).reshape(seq, (bsz * d) // 128, 128)
    w_bd = jnp.kron(jnp.eye(pack, dtype=W_curv.dtype), W_curv.T)
    b_col = jnp.tile(b_curv, pack).reshape(pack * d, 1)
    emb_t, curv_t = _project_tc(emb4, w_bd, b_col, seq, bsz, d)
    return (
        jnp.transpose(emb_t, (2, 0, 1)),
        jnp.transpose(curv_t, (2, 0, 1)),
    )
